# baseline (speedup 1.0000x reference)
import functools

import jax
import jax.numpy as jnp
from jax import lax
from jax.experimental import pallas as pl
from jax.experimental.pallas import tpu as pltpu
from jax.experimental.pallas import tpu_sc as plsc

N_CAT = 1000
L = 16
NC = 2
NS = 16
NW = NC * NS
T_SC = 2048
CI = 2
BR = 128


def _one_hot_sc(x_tail, n0, n1):
    rows_w = T_SC // NW
    n_chunks = rows_w // CI
    rpc = CI * n1
    mesh = plsc.VectorSubcoreMesh(core_axis_name="c", subcore_axis_name="s")

    @functools.partial(
        pl.kernel,
        out_type=jax.ShapeDtypeStruct((n0, n1, N_CAT), jnp.float32),
        mesh=mesh,
        scratch_types=[
            pltpu.VMEM((rows_w * n1,), jnp.int32),
            pltpu.VMEM((CI, n1, N_CAT), jnp.float32),
        ],
        compiler_params=pltpu.CompilerParams(
            needs_layout_passes=False, use_tc_tiling_on_sc=True
        ),
    )
    def body(x_hbm, out_hbm, idx_v, buf_v):
        wid = lax.axis_index("s") * NC + lax.axis_index("c")
        i_base = (n0 - T_SC) + wid * rows_w

        pltpu.sync_copy(x_hbm.at[pl.ds(wid * rows_w * n1, rows_w * n1)], idx_v)

        zeros = jnp.zeros((L,), jnp.float32)
        ones = jnp.ones((L,), jnp.float32)
        lane = lax.iota(jnp.int32, L)

        def zero_body(w, _):
            p = w * L + lane
            plsc.store_scatter(
                buf_v,
                [p // (n1 * N_CAT), (p // N_CAT) % n1, p % N_CAT],
                zeros,
            )
            return 0

        lax.fori_loop(0, (CI * n1 * N_CAT) // L, zero_body, 0)

        n_full, tail = divmod(rpc, L)

        def scatter_vals(k, vals):
            for g in range(n_full + (1 if tail else 0)):
                cols = idx_v[pl.ds(k * rpc + g * L, L)]
                r = g * L + lane
                idxs = [r // n1, r % n1, cols]
                if g < n_full:
                    plsc.store_scatter(buf_v, idxs, vals)
                else:
                    plsc.store_scatter(buf_v, idxs, vals, mask=lane < tail)

        def chunk_body(k, _):
            scatter_vals(k, ones)
            pltpu.sync_copy(buf_v, out_hbm.at[pl.ds(i_base + k * CI, CI)])
            scatter_vals(k, zeros)
            return 0

        lax.fori_loop(0, n_chunks, chunk_body, 0)

    return body(x_tail)


def _one_hot_tc(x_head, buf):
    n0, n1, _ = buf.shape
    nb = (n0 - T_SC) // BR

    def body(x_ref, buf_ref, o_ref):
        del buf_ref
        idx = x_ref[...]
        iota = lax.broadcasted_iota(jnp.int32, (BR, n1, N_CAT), 2)
        o_ref[...] = (iota == idx[:, :, None]).astype(jnp.float32)

    return pl.pallas_call(
        body,
        grid=(nb,),
        in_specs=[
            pl.BlockSpec((BR, n1), lambda i: (i, 0)),
            pl.BlockSpec(memory_space=pl.ANY),
        ],
        out_specs=pl.BlockSpec((BR, n1, N_CAT), lambda i: (i, 0, 0)),
        out_shape=jax.ShapeDtypeStruct((n0, n1, N_CAT), jnp.float32),
        input_output_aliases={1: 0},
    )(x_head, buf)


def kernel(x, eye):
    n0, n1 = x.shape
    xi = x.astype(jnp.int32)
    x_tail = xi[n0 - T_SC:].reshape(T_SC * n1)
    buf = _one_hot_sc(x_tail, n0, n1)
    return _one_hot_tc(xi[: n0 - T_SC], buf)


# speedup vs baseline: 1.0052x; 1.0052x over previous
import functools

import jax
import jax.numpy as jnp
from jax import lax
from jax.experimental import pallas as pl
from jax.experimental.pallas import tpu as pltpu
from jax.experimental.pallas import tpu_sc as plsc

N_CAT = 1000
L = 16
NC = 2
NS = 16
NW = NC * NS
T_SC = 4096
CI = 2
BR = 128


def _one_hot_sc(x_tail, n0, n1):
    rows_w = T_SC // NW
    n_chunks = rows_w // CI
    rpc = CI * n1
    mesh = plsc.VectorSubcoreMesh(core_axis_name="c", subcore_axis_name="s")

    @functools.partial(
        pl.kernel,
        out_type=jax.ShapeDtypeStruct((n0, n1, N_CAT), jnp.float32),
        mesh=mesh,
        scratch_types=[
            pltpu.VMEM((rows_w * n1,), jnp.int32),
            pltpu.VMEM((CI, n1, N_CAT), jnp.float32),
        ],
        compiler_params=pltpu.CompilerParams(
            needs_layout_passes=False, use_tc_tiling_on_sc=True
        ),
    )
    def body(x_hbm, out_hbm, idx_v, buf_v):
        wid = lax.axis_index("s") * NC + lax.axis_index("c")
        i_base = (n0 - T_SC) + wid * rows_w

        pltpu.sync_copy(x_hbm.at[pl.ds(wid * rows_w * n1, rows_w * n1)], idx_v)

        zeros = jnp.zeros((L,), jnp.float32)
        ones = jnp.ones((L,), jnp.float32)
        lane = lax.iota(jnp.int32, L)

        def zero_body(w, _):
            p = w * L + lane
            plsc.store_scatter(
                buf_v,
                [p // (n1 * N_CAT), (p // N_CAT) % n1, p % N_CAT],
                zeros,
            )
            return 0

        lax.fori_loop(0, (CI * n1 * N_CAT) // L, zero_body, 0)

        n_full, tail = divmod(rpc, L)

        def scatter_vals(k, vals):
            for g in range(n_full + (1 if tail else 0)):
                cols = idx_v[pl.ds(k * rpc + g * L, L)]
                r = g * L + lane
                idxs = [r // n1, r % n1, cols]
                if g < n_full:
                    plsc.store_scatter(buf_v, idxs, vals)
                else:
                    plsc.store_scatter(buf_v, idxs, vals, mask=lane < tail)

        def chunk_body(k, _):
            scatter_vals(k, ones)
            pltpu.sync_copy(buf_v, out_hbm.at[pl.ds(i_base + k * CI, CI)])
            scatter_vals(k, zeros)
            return 0

        lax.fori_loop(0, n_chunks, chunk_body, 0)

    return body(x_tail)


def _one_hot_tc(x_head, buf):
    n0, n1, _ = buf.shape
    nb = (n0 - T_SC) // BR

    def body(x_ref, buf_ref, o_ref):
        del buf_ref
        idx = x_ref[...]
        iota = lax.broadcasted_iota(jnp.int32, (BR, n1, N_CAT), 2)
        o_ref[...] = (iota == idx[:, :, None]).astype(jnp.float32)

    return pl.pallas_call(
        body,
        grid=(nb,),
        in_specs=[
            pl.BlockSpec((BR, n1), lambda i: (i, 0)),
            pl.BlockSpec(memory_space=pl.ANY),
        ],
        out_specs=pl.BlockSpec((BR, n1, N_CAT), lambda i: (i, 0, 0)),
        out_shape=jax.ShapeDtypeStruct((n0, n1, N_CAT), jnp.float32),
        input_output_aliases={1: 0},
    )(x_head, buf)


def kernel(x, eye):
    n0, n1 = x.shape
    xi = x.astype(jnp.int32)
    x_tail = xi[n0 - T_SC:].reshape(T_SC * n1)
    buf = _one_hot_sc(x_tail, n0, n1)
    if T_SC == n0:
        return buf
    return _one_hot_tc(xi[: n0 - T_SC], buf)


# speedup vs baseline: 1.0653x; 1.0598x over previous
import functools

import jax
import jax.numpy as jnp
from jax import lax
from jax.experimental import pallas as pl
from jax.experimental.pallas import tpu as pltpu
from jax.experimental.pallas import tpu_sc as plsc

N_CAT = 1000
L = 16
NC = 2
NS = 16
NW = NC * NS
T_SC = 4096
CI = 1
NBUF = 2
BR = 128


def _one_hot_sc(x_tail, n0, n1):
    rows_w = T_SC // NW
    n_chunks = rows_w // CI
    rpc = CI * n1
    mesh = plsc.VectorSubcoreMesh(core_axis_name="c", subcore_axis_name="s")

    @functools.partial(
        pl.kernel,
        out_type=jax.ShapeDtypeStruct((n0, n1, N_CAT), jnp.float32),
        mesh=mesh,
        scratch_types=[
            pltpu.VMEM((rows_w * n1,), jnp.int32),
            [pltpu.VMEM((CI, n1, N_CAT), jnp.float32)] * NBUF,
            [pltpu.SemaphoreType.DMA] * NBUF,
        ],
        compiler_params=pltpu.CompilerParams(
            needs_layout_passes=False, use_tc_tiling_on_sc=True
        ),
    )
    def body(x_hbm, out_hbm, idx_v, bufs, sems):
        wid = lax.axis_index("s") * NC + lax.axis_index("c")
        i_base = (n0 - T_SC) + wid * rows_w

        pltpu.sync_copy(x_hbm.at[pl.ds(wid * rows_w * n1, rows_w * n1)], idx_v)

        zeros = jnp.zeros((L,), jnp.float32)
        ones = jnp.ones((L,), jnp.float32)
        lane = lax.iota(jnp.int32, L)

        def zero_body(w, _):
            p = w * L + lane
            idxs = [p // (n1 * N_CAT), (p // N_CAT) % n1, p % N_CAT]
            for b in range(NBUF):
                plsc.store_scatter(bufs[b], idxs, zeros)
            return 0

        lax.fori_loop(0, (CI * n1 * N_CAT) // L, zero_body, 0)

        n_full, tail = divmod(rpc, L)

        def scatter_vals(b, k, vals):
            for g in range(n_full + (1 if tail else 0)):
                cols = idx_v[pl.ds(k * rpc + g * L, L)]
                r = g * L + lane
                idxs = [r // n1, r % n1, cols]
                if g < n_full:
                    plsc.store_scatter(bufs[b], idxs, vals)
                else:
                    plsc.store_scatter(bufs[b], idxs, vals, mask=lane < tail)

        def dma(b, k):
            return pltpu.make_async_copy(
                bufs[b], out_hbm.at[pl.ds(i_base + k * CI, CI)], sems[b]
            )

        for b in range(NBUF):
            scatter_vals(b, b, ones)
            dma(b, b).start()

        def group_body(g, _):
            for b in range(NBUF):
                k = g * NBUF + b
                dma(b, k - NBUF).wait()
                scatter_vals(b, k - NBUF, zeros)
                scatter_vals(b, k, ones)
                dma(b, k).start()
            return 0

        lax.fori_loop(1, n_chunks // NBUF, group_body, 0)

        for b in range(NBUF):
            dma(b, n_chunks - NBUF + b).wait()

    return body(x_tail)


def _one_hot_tc(x_head, buf):
    n0, n1, _ = buf.shape
    nb = (n0 - T_SC) // BR

    def body(x_ref, buf_ref, o_ref):
        del buf_ref
        idx = x_ref[...]
        iota = lax.broadcasted_iota(jnp.int32, (BR, n1, N_CAT), 2)
        o_ref[...] = (iota == idx[:, :, None]).astype(jnp.float32)

    return pl.pallas_call(
        body,
        grid=(nb,),
        in_specs=[
            pl.BlockSpec((BR, n1), lambda i: (i, 0)),
            pl.BlockSpec(memory_space=pl.ANY),
        ],
        out_specs=pl.BlockSpec((BR, n1, N_CAT), lambda i: (i, 0, 0)),
        out_shape=jax.ShapeDtypeStruct((n0, n1, N_CAT), jnp.float32),
        input_output_aliases={1: 0},
    )(x_head, buf)


def kernel(x, eye):
    n0, n1 = x.shape
    xi = x.astype(jnp.int32)
    x_tail = xi[n0 - T_SC:].reshape(T_SC * n1)
    buf = _one_hot_sc(x_tail, n0, n1)
    if T_SC == n0:
        return buf
    return _one_hot_tc(xi[: n0 - T_SC], buf)
